# baseline (speedup 1.0000x reference)
import functools

import jax
import jax.numpy as jnp
from jax.experimental import pallas as pl
from jax.experimental.pallas import tpu as pltpu


def _pick_b_tile(B):
    for tb in (8, 4, 2):
        if B % tb == 0:
            return tb
    return B


def _fused_kernel(x_ref, w1_ref, b1_ref, s_ref, t_ref, w2t_ref, b2_ref,
                  out_ref, *, inv_hw):
    feat = jnp.sum(x_ref[...], axis=0, dtype=jnp.float32) * inv_hw
    h = jnp.dot(feat, w1_ref[...], preferred_element_type=jnp.float32)
    h = h + b1_ref[...]
    h = jnp.maximum(h * s_ref[...] + t_ref[...], 0.0)
    scores_t = jnp.dot(w2t_ref[...], h.T, preferred_element_type=jnp.float32)
    out_ref[...] = scores_t.T + b2_ref[...]


@jax.jit
def _forward(x, w1, b1, bn_scale, bn_shift, w2, b2):
    B, C, H, W = x.shape
    HW = H * W
    D1 = w1.shape[1]
    NC = w2.shape[1]
    xt = jnp.transpose(x, (2, 3, 0, 1)).reshape(HW, B, C)
    w2t = jnp.transpose(w2)
    tile_b = _pick_b_tile(B)
    grid = (B // tile_b,)
    body = functools.partial(_fused_kernel, inv_hw=1.0 / float(HW))
    return pl.pallas_call(
        body,
        grid=grid,
        in_specs=[
            pl.BlockSpec((HW, tile_b, C), lambda i: (0, i, 0)),
            pl.BlockSpec((C, D1), lambda i: (0, 0)),
            pl.BlockSpec((1, D1), lambda i: (0, 0)),
            pl.BlockSpec((1, D1), lambda i: (0, 0)),
            pl.BlockSpec((1, D1), lambda i: (0, 0)),
            pl.BlockSpec((NC, D1), lambda i: (0, 0)),
            pl.BlockSpec((1, NC), lambda i: (0, 0)),
        ],
        out_specs=pl.BlockSpec((tile_b, NC), lambda i: (i, 0)),
        out_shape=jax.ShapeDtypeStruct((B, NC), jnp.float32),
        compiler_params=pltpu.CompilerParams(
            dimension_semantics=("parallel",)),
        cost_estimate=pl.CostEstimate(
            flops=B * C * HW + 2 * B * C * D1 + 2 * B * D1 * NC,
            transcendentals=0,
            bytes_accessed=(B * C * HW * 4 + C * D1 * 4 + 3 * D1 * 4
                            + D1 * NC * 4 + NC * 4 + B * NC * 4)),
    )(xt, w1, b1, bn_scale, bn_shift, w2t, b2)


def kernel(x, w1, b1, bn_scale, bn_shift, w2, b2):
    return _forward(x, w1, b1, bn_scale, bn_shift, w2, b2)


# speedup vs baseline: 1.2941x; 1.2941x over previous
import functools

import jax
import jax.numpy as jnp
from jax.experimental import pallas as pl
from jax.experimental.pallas import tpu as pltpu


def _pick_b_tile(B):
    for tb in (32, 16, 8, 4, 2):
        if B % tb == 0:
            return tb
    return B


def _fused_kernel(x_ref, w1_ref, b1_ref, s_ref, t_ref, w2t_ref, b2_ref,
                  out_ref, *, inv_hw):
    feat = jnp.sum(x_ref[...], axis=0, dtype=jnp.float32) * inv_hw
    h = jnp.dot(feat, w1_ref[...], preferred_element_type=jnp.float32)
    h = h + b1_ref[...]
    h = jnp.maximum(h * s_ref[...] + t_ref[...], 0.0)
    scores_t = jnp.dot(w2t_ref[...], h.T, preferred_element_type=jnp.float32)
    out_ref[...] = scores_t.T + b2_ref[...]


@jax.jit
def _forward(x, w1, b1, bn_scale, bn_shift, w2, b2):
    B, C, H, W = x.shape
    HW = H * W
    D1 = w1.shape[1]
    NC = w2.shape[1]
    xt = jnp.transpose(x, (2, 3, 0, 1)).reshape(HW, B, C)
    w2t = jnp.transpose(w2)
    tile_b = _pick_b_tile(B)
    grid = (B // tile_b,)
    body = functools.partial(_fused_kernel, inv_hw=1.0 / float(HW))
    return pl.pallas_call(
        body,
        grid=grid,
        in_specs=[
            pl.BlockSpec((HW, tile_b, C), lambda i: (0, i, 0)),
            pl.BlockSpec((C, D1), lambda i: (0, 0)),
            pl.BlockSpec((1, D1), lambda i: (0, 0)),
            pl.BlockSpec((1, D1), lambda i: (0, 0)),
            pl.BlockSpec((1, D1), lambda i: (0, 0)),
            pl.BlockSpec((NC, D1), lambda i: (0, 0)),
            pl.BlockSpec((1, NC), lambda i: (0, 0)),
        ],
        out_specs=pl.BlockSpec((tile_b, NC), lambda i: (i, 0)),
        out_shape=jax.ShapeDtypeStruct((B, NC), jnp.float32),
        compiler_params=pltpu.CompilerParams(
            dimension_semantics=("parallel",)),
        cost_estimate=pl.CostEstimate(
            flops=B * C * HW + 2 * B * C * D1 + 2 * B * D1 * NC,
            transcendentals=0,
            bytes_accessed=(B * C * HW * 4 + C * D1 * 4 + 3 * D1 * 4
                            + D1 * NC * 4 + NC * 4 + B * NC * 4)),
    )(xt, w1, b1, bn_scale, bn_shift, w2t, b2)


def kernel(x, w1, b1, bn_scale, bn_shift, w2, b2):
    return _forward(x, w1, b1, bn_scale, bn_shift, w2, b2)
